# Initial kernel scaffold; baseline (speedup 1.0000x reference)
import jax
import jax.numpy as jnp
from jax.experimental import pallas as pl


def kernel(x, edge_index, W, b):
    raise NotImplementedError("write your pallas kernel here")


# speedup vs baseline: 7.5401x; 7.5401x over previous
import functools

import jax
import jax.numpy as jnp
from jax import lax
from jax.experimental import pallas as pl
from jax.experimental.pallas import tpu as pltpu
from jax.experimental.pallas import tpu_sc as plsc

N_NODES = 10000
N_EDGES = 320000
D = 128

NC = 2
NS = 16
NW = NC * NS
EDGES_PER_W = N_EDGES // NW
CHUNK = 80
NCHUNK = EDGES_PER_W // CHUNK
ACC_ROWS = 10240
ROWS_PER_TILE = ACC_ROWS // NS


def _sc_segment_sum(x, src_r, dst_r):
    mesh = plsc.VectorSubcoreMesh(
        core_axis_name="c", subcore_axis_name="s", num_cores=NC, num_subcores=NS
    )

    @functools.partial(
        pl.kernel,
        out_type=jax.ShapeDtypeStruct((NC, ACC_ROWS, D), jnp.float32),
        mesh=mesh,
        scratch_types=[
            pltpu.VMEM((NCHUNK, CHUNK), jnp.int32),
            pltpu.VMEM((NCHUNK, CHUNK), jnp.int32),
            pltpu.VMEM((CHUNK, D), jnp.float32),
            pltpu.VMEM_SHARED((ACC_ROWS, D), jnp.float32),
            pltpu.SemaphoreType.DMA,
        ],
    )
    def k(x_hbm, src_hbm, dst_hbm, out_hbm, src_v, dst_v, rows_v, acc, sem):
        c = lax.axis_index("c")
        s = lax.axis_index("s")
        wid = s * NC + c

        zeros16 = jnp.zeros((16,), jnp.float32)

        def zero_row(r, _):
            for kk in range(D // 16):
                rows_v[r, pl.ds(kk * 16, 16)] = zeros16
            return _

        lax.fori_loop(0, CHUNK, zero_row, None)
        for blk in range(ROWS_PER_TILE // CHUNK):
            pltpu.sync_copy(rows_v, acc.at[pl.ds(s * ROWS_PER_TILE + blk * CHUNK, CHUNK)])
        plsc.subcore_barrier()

        pltpu.sync_copy(src_hbm.at[wid], src_v)
        pltpu.sync_copy(dst_hbm.at[wid], dst_v)

        def body(j, _):
            pltpu.async_copy(x_hbm.at[src_v.at[j]], rows_v, sem).wait()
            pltpu.sync_copy(rows_v, acc.at[dst_v.at[j]], add=True)
            return _

        lax.fori_loop(0, NCHUNK, body, None)
        plsc.subcore_barrier()

        pltpu.sync_copy(
            acc.at[pl.ds(s * ROWS_PER_TILE, ROWS_PER_TILE)],
            out_hbm.at[c, pl.ds(s * ROWS_PER_TILE, ROWS_PER_TILE)],
        )

    return k(x, src_r, dst_r)


def _tc_linear(p0, p1, Wt, b2d):
    BLK = 1000

    def body(p0_ref, p1_ref, wt_ref, b_ref, out_ref):
        h = p0_ref[...] + p1_ref[...]
        out_ref[...] = (
            jnp.dot(h, wt_ref[...], preferred_element_type=jnp.float32) + b_ref[...]
        )

    return pl.pallas_call(
        body,
        out_shape=jax.ShapeDtypeStruct((N_NODES, D), jnp.float32),
        grid=(N_NODES // BLK,),
        in_specs=[
            pl.BlockSpec((BLK, D), lambda i: (i, 0)),
            pl.BlockSpec((BLK, D), lambda i: (i, 0)),
            pl.BlockSpec((D, D), lambda i: (0, 0)),
            pl.BlockSpec((1, D), lambda i: (0, 0)),
        ],
        out_specs=pl.BlockSpec((BLK, D), lambda i: (i, 0)),
    )(p0, p1, Wt, b2d)


@jax.jit
def kernel(x, edge_index, W, b):
    src = edge_index[0].astype(jnp.int32).reshape(NW, NCHUNK, CHUNK)
    dst = edge_index[1].astype(jnp.int32).reshape(NW, NCHUNK, CHUNK)
    partials = _sc_segment_sum(x, src, dst)
    p0 = partials[0, :N_NODES]
    p1 = partials[1, :N_NODES]
    return _tc_linear(p0, p1, W.T, b.reshape(1, D))


# speedup vs baseline: 11.7466x; 1.5579x over previous
import functools

import jax
import jax.numpy as jnp
from jax import lax
from jax.experimental import pallas as pl
from jax.experimental.pallas import tpu as pltpu
from jax.experimental.pallas import tpu_sc as plsc

N_NODES = 10000
N_EDGES = 320000
D = 128

NC = 2
NS = 16
NW = NC * NS
EDGES_PER_W = N_EDGES // NW
CHUNK = 80
NCHUNK = EDGES_PER_W // CHUNK
NBUF = 2
ACC_ROWS = 10240
ROWS_PER_TILE = ACC_ROWS // NS


def _sc_segment_sum(x, src_r, dst_r):
    mesh = plsc.VectorSubcoreMesh(
        core_axis_name="c", subcore_axis_name="s", num_cores=NC, num_subcores=NS
    )

    @functools.partial(
        pl.kernel,
        out_type=jax.ShapeDtypeStruct((NC, ACC_ROWS, D), jnp.float32),
        mesh=mesh,
        scratch_types=[
            pltpu.VMEM((EDGES_PER_W,), jnp.int32),
            pltpu.VMEM((NCHUNK, CHUNK), jnp.int32),
            pltpu.VMEM((CHUNK, D), jnp.float32),
            pltpu.VMEM((CHUNK, D), jnp.float32),
            pltpu.VMEM_SHARED((ACC_ROWS, D), jnp.float32),
            pltpu.SemaphoreType.DMA((NBUF,)),
            pltpu.SemaphoreType.DMA,
        ],
    )
    def k(x_hbm, src_hbm, dst_hbm, out_hbm, src_v, dst_v, rows0, rows1, acc, gsems, isem):
        bufs = (rows0, rows1)
        c = lax.axis_index("c")
        s = lax.axis_index("s")
        wid = s * NC + c

        pltpu.async_copy(src_hbm.at[wid], src_v, isem)
        pltpu.async_copy(dst_hbm.at[wid], dst_v, isem)

        zeros16 = jnp.zeros((16,), jnp.float32)

        def zero_row(r, _):
            for kk in range(D // 16):
                rows0[r, pl.ds(kk * 16, 16)] = zeros16
            return _

        lax.fori_loop(0, CHUNK, zero_row, None)
        for blk in range(ROWS_PER_TILE // CHUNK):
            pltpu.sync_copy(
                rows0, acc.at[pl.ds(s * ROWS_PER_TILE + blk * CHUNK, CHUNK)]
            )
        pltpu.make_async_copy(src_hbm.at[wid], src_v, isem).wait()
        pltpu.make_async_copy(dst_hbm.at[wid], dst_v, isem).wait()
        plsc.subcore_barrier()

        def start_gather(j, bb):
            pltpu.async_copy(
                x_hbm.at[src_v.at[pl.ds(j * CHUNK, CHUNK)]], bufs[bb], gsems.at[bb]
            )

        def wait_gather(bb):
            pltpu.make_async_copy(
                x_hbm.at[src_v.at[pl.ds(0, CHUNK)]], bufs[bb], gsems.at[bb]
            ).wait()

        for bb in range(NBUF):
            start_gather(bb, bb)

        def group(g, _):
            for bb in range(NBUF):
                j = g * NBUF + bb
                wait_gather(bb)
                pltpu.sync_copy(bufs[bb], acc.at[dst_v.at[j]], add=True)

                @pl.when(j + NBUF < NCHUNK)
                def _():
                    start_gather(j + NBUF, bb)

            return _

        lax.fori_loop(0, NCHUNK // NBUF, group, None)
        for j in range((NCHUNK // NBUF) * NBUF, NCHUNK):
            bb = j % NBUF
            wait_gather(bb)
            pltpu.sync_copy(bufs[bb], acc.at[dst_v.at[j]], add=True)
        plsc.subcore_barrier()

        pltpu.sync_copy(
            acc.at[pl.ds(s * ROWS_PER_TILE, ROWS_PER_TILE)],
            out_hbm.at[c, pl.ds(s * ROWS_PER_TILE, ROWS_PER_TILE)],
        )

    return k(x, src_r, dst_r)


def _tc_linear(p0, p1, Wt, b2d):
    BLK = 1000

    def body(p0_ref, p1_ref, wt_ref, b_ref, out_ref):
        h = p0_ref[...] + p1_ref[...]
        out_ref[...] = (
            jnp.dot(h, wt_ref[...], preferred_element_type=jnp.float32) + b_ref[...]
        )

    return pl.pallas_call(
        body,
        out_shape=jax.ShapeDtypeStruct((N_NODES, D), jnp.float32),
        grid=(N_NODES // BLK,),
        in_specs=[
            pl.BlockSpec((BLK, D), lambda i: (i, 0)),
            pl.BlockSpec((BLK, D), lambda i: (i, 0)),
            pl.BlockSpec((D, D), lambda i: (0, 0)),
            pl.BlockSpec((1, D), lambda i: (0, 0)),
        ],
        out_specs=pl.BlockSpec((BLK, D), lambda i: (i, 0)),
    )(p0, p1, Wt, b2d)


@jax.jit
def kernel(x, edge_index, W, b):
    src = edge_index[0].astype(jnp.int32).reshape(NW, EDGES_PER_W)
    dst = edge_index[1].astype(jnp.int32).reshape(NW, NCHUNK, CHUNK)
    partials = _sc_segment_sum(x, src, dst)
    p0 = partials[0, :N_NODES]
    p1 = partials[1, :N_NODES]
    return _tc_linear(p0, p1, W.T, b.reshape(1, D))
